# baseline (device time: 54746 ns/iter reference)
import os

import jax
import jax.numpy as jnp
from jax import lax
from jax.experimental import pallas as pl
from jax.experimental.pallas import tpu as pltpu

N_DEV = 4
OFFS = (2, 1, 3, 0)
_NO_COMM = os.environ.get("KERNEL_NO_COMM", "0") == "1"


def kernel(x, w_mat):
    m_per, k = x.shape
    n = w_mat.shape[1]
    n_per = n // N_DEV

    def body(x_ref, w_hbm, out_ref, xbf, wstage, ybuf, rbuf,
             amax_sbuf, amax_rbuf,
             wsems, dsend_sems, drecv_sems, asend_sems, arecv_sems):
        my = lax.axis_index("i")

        def start_w(j):
            p = (my + OFFS[j]) % N_DEV
            cp = pltpu.make_async_copy(
                w_hbm.at[:, pl.ds(p * n_per, n_per)],
                wstage.at[j % 2],
                wsems.at[j % 2],
            )
            cp.start()
            return cp

        copies = {0: start_w(0), 1: start_w(1)}

        xbf[...] = x_ref[...].astype(jnp.bfloat16)

        data_rdmas = {}
        amax_rdmas = []
        amax = jnp.float32(0.0)
        m_half = m_per // 2
        for j in range(N_DEV):
            d = OFFS[j]
            with jax.named_scope(f"wwait#j={j}"):
                copies[j].wait()
            wb = wstage[j % 2].astype(jnp.bfloat16)
            subs = []
            for h in range(2):
                rows = pl.ds(h * m_half, m_half)
                with jax.named_scope(f"dot#j={j}h={h}"):
                    yb = jnp.dot(xbf[rows, :], wb,
                                 preferred_element_type=jnp.float32)
                    amax = jnp.maximum(amax, jnp.max(jnp.abs(yb)))
                    ybuf[j, rows] = yb.astype(jnp.bfloat16)
            if d != 0 and not _NO_COMM:
                rdma = pltpu.make_async_remote_copy(
                    src_ref=ybuf.at[j],
                    dst_ref=rbuf.at[d - 1],
                    send_sem=dsend_sems.at[d - 1, 0],
                    recv_sem=drecv_sems.at[d - 1, 0],
                    device_id=((my + d) % N_DEV,),
                    device_id_type=pl.DeviceIdType.MESH,
                )
                rdma.start()
                subs.append(rdma)
            if subs:
                data_rdmas[d] = subs
            if j + 2 < N_DEV:
                copies[j + 2] = start_w(j + 2)

        amax_sbuf[...] = jnp.full((8, 128), amax, jnp.float32)
        amax_rbuf[0] = amax_sbuf[...]
        for d in range(1, N_DEV) if not _NO_COMM else []:
            rdma = pltpu.make_async_remote_copy(
                src_ref=amax_sbuf,
                dst_ref=amax_rbuf.at[d],
                send_sem=asend_sems.at[d - 1],
                recv_sem=arecv_sems.at[d - 1],
                device_id=((my + d) % N_DEV,),
                device_id_type=pl.DeviceIdType.MESH,
            )
            rdma.start()
            amax_rdmas.append(rdma)

        with jax.named_scope("amax_wait"):
            for rdma in amax_rdmas:
                rdma.wait_recv()

        g_amax = jnp.max(amax_rbuf[...])
        inv_scale = 127.0 / g_amax
        scale = g_amax / 127.0

        def qdq(v):
            q = jnp.clip(jnp.round(v.astype(jnp.float32) * inv_scale),
                         -127.0, 127.0)
            return q * scale

        with jax.named_scope("own_qdq"):
            out_ref[pl.ds(my * m_per, m_per), :] = qdq(
                ybuf[3]).astype(out_ref.dtype)

        for d in ((2, 1, 3) if not _NO_COMM else ()):
            origin = (my - d) % N_DEV
            for h, sub in enumerate(data_rdmas[d]):
                with jax.named_scope(f"drain_wait#d={d}h={h}"):
                    sub.wait_recv()
            with jax.named_scope(f"drain_qdq#d={d}"):
                out_ref[pl.ds(origin * m_per, m_per), :] = (
                    qdq(rbuf[d - 1]).astype(out_ref.dtype))

        if not _NO_COMM:
            with jax.named_scope("send_waits"):
                for d in (2, 1, 3):
                    for sub in data_rdmas[d]:
                        sub.wait_send()
                for rdma in amax_rdmas:
                    rdma.wait_send()

    return pl.pallas_call(
        body,
        out_shape=jax.ShapeDtypeStruct((N_DEV * m_per, n_per), jnp.bfloat16),
        in_specs=[
            pl.BlockSpec(memory_space=pltpu.VMEM),
            pl.BlockSpec(memory_space=pltpu.MemorySpace.HBM),
        ],
        out_specs=pl.BlockSpec(memory_space=pltpu.VMEM),
        scratch_shapes=[
            pltpu.VMEM((m_per, k), jnp.bfloat16),
            pltpu.VMEM((2, k, n_per), jnp.float32),
            pltpu.VMEM((N_DEV, m_per, n_per), jnp.bfloat16),
            pltpu.VMEM((N_DEV - 1, m_per, n_per), jnp.bfloat16),
            pltpu.VMEM((8, 128), jnp.float32),
            pltpu.VMEM((N_DEV, 8, 128), jnp.float32),
            pltpu.SemaphoreType.DMA((2,)),
            pltpu.SemaphoreType.DMA((N_DEV - 1, 2)),
            pltpu.SemaphoreType.DMA((N_DEV - 1, 2)),
            pltpu.SemaphoreType.DMA((N_DEV - 1,)),
            pltpu.SemaphoreType.DMA((N_DEV - 1,)),
        ],
        compiler_params=pltpu.CompilerParams(
            vmem_limit_bytes=63 * 1024 * 1024,
        ),
    )(x, w_mat)


# device time: 54148 ns/iter; 1.0110x vs baseline; 1.0110x over previous
import os

import jax
import jax.numpy as jnp
from jax import lax
from jax.experimental import pallas as pl
from jax.experimental.pallas import tpu as pltpu

N_DEV = 4
OFFS = (2, 1, 3, 0)
_NO_COMM = os.environ.get("KERNEL_NO_COMM", "0") == "1"


def kernel(x, w_mat):
    m_per, k = x.shape
    n = w_mat.shape[1]
    n_per = n // N_DEV

    def body(x_ref, w_hbm, out_ref, xbf, wstage, ybuf, rbuf,
             amax_sbuf, amax_rbuf,
             wsems, dsend_sems, drecv_sems, asend_sems, arecv_sems):
        my = lax.axis_index("i")

        def start_w(j):
            p = (my + OFFS[j]) % N_DEV
            cp = pltpu.make_async_copy(
                w_hbm.at[:, pl.ds(p * n_per, n_per)],
                wstage.at[j % 2],
                wsems.at[j % 2],
            )
            cp.start()
            return cp

        copies = {0: start_w(0), 1: start_w(1)}

        xbf[...] = x_ref[...].astype(jnp.bfloat16)

        data_rdmas = {}
        amax_rdmas = []
        amax = jnp.float32(0.0)
        m_half = m_per // 2
        for j in range(N_DEV):
            d = OFFS[j]
            with jax.named_scope(f"wwait#j={j}"):
                copies[j].wait()
            wb = wstage[j % 2].astype(jnp.bfloat16)
            subs = []
            for h in range(2):
                rows = pl.ds(h * m_half, m_half)
                with jax.named_scope(f"dot#j={j}h={h}"):
                    yb = jnp.dot(xbf[rows, :], wb,
                                 preferred_element_type=jnp.float32)
                    amax = jnp.maximum(amax, jnp.max(jnp.abs(yb)))
                    ybuf[j, rows] = yb.astype(jnp.bfloat16)
                if d != 0 and not _NO_COMM:
                    rdma = pltpu.make_async_remote_copy(
                        src_ref=ybuf.at[j, rows],
                        dst_ref=rbuf.at[d - 1, rows],
                        send_sem=dsend_sems.at[d - 1, h],
                        recv_sem=drecv_sems.at[d - 1, h],
                        device_id=((my + d) % N_DEV,),
                        device_id_type=pl.DeviceIdType.MESH,
                    )
                    rdma.start()
                    subs.append(rdma)
            if subs:
                data_rdmas[d] = subs
            if j + 2 < N_DEV:
                copies[j + 2] = start_w(j + 2)

        amax_sbuf[...] = jnp.full((8, 128), amax, jnp.float32)
        amax_rbuf[0] = amax_sbuf[...]
        for d in range(1, N_DEV) if not _NO_COMM else []:
            rdma = pltpu.make_async_remote_copy(
                src_ref=amax_sbuf,
                dst_ref=amax_rbuf.at[d],
                send_sem=asend_sems.at[d - 1],
                recv_sem=arecv_sems.at[d - 1],
                device_id=((my + d) % N_DEV,),
                device_id_type=pl.DeviceIdType.MESH,
            )
            rdma.start()
            amax_rdmas.append(rdma)

        with jax.named_scope("amax_wait"):
            for rdma in amax_rdmas:
                rdma.wait_recv()

        g_amax = jnp.max(amax_rbuf[...])
        inv_scale = 127.0 / g_amax
        scale = g_amax / 127.0

        def qdq(v):
            q = jnp.clip(jnp.round(v.astype(jnp.float32) * inv_scale),
                         -127.0, 127.0)
            return q * scale

        with jax.named_scope("own_qdq"):
            out_ref[pl.ds(my * m_per, m_per), :] = qdq(
                ybuf[3]).astype(out_ref.dtype)

        for d in ((2, 1, 3) if not _NO_COMM else ()):
            origin = (my - d) % N_DEV
            for h, sub in enumerate(data_rdmas[d]):
                with jax.named_scope(f"drain_wait#d={d}h={h}"):
                    sub.wait_recv()
                with jax.named_scope(f"drain_qdq#d={d}h={h}"):
                    rows = pl.ds(h * m_half, m_half)
                    out_ref[pl.ds(origin * m_per + h * m_half, m_half), :] = (
                        qdq(rbuf[d - 1, rows]).astype(out_ref.dtype))

        if not _NO_COMM:
            with jax.named_scope("send_waits"):
                for d in (2, 1, 3):
                    for sub in data_rdmas[d]:
                        sub.wait_send()
                for rdma in amax_rdmas:
                    rdma.wait_send()

    return pl.pallas_call(
        body,
        out_shape=jax.ShapeDtypeStruct((N_DEV * m_per, n_per), jnp.bfloat16),
        in_specs=[
            pl.BlockSpec(memory_space=pltpu.VMEM),
            pl.BlockSpec(memory_space=pltpu.MemorySpace.HBM),
        ],
        out_specs=pl.BlockSpec(memory_space=pltpu.VMEM),
        scratch_shapes=[
            pltpu.VMEM((m_per, k), jnp.bfloat16),
            pltpu.VMEM((2, k, n_per), jnp.float32),
            pltpu.VMEM((N_DEV, m_per, n_per), jnp.bfloat16),
            pltpu.VMEM((N_DEV - 1, m_per, n_per), jnp.bfloat16),
            pltpu.VMEM((8, 128), jnp.float32),
            pltpu.VMEM((N_DEV, 8, 128), jnp.float32),
            pltpu.SemaphoreType.DMA((2,)),
            pltpu.SemaphoreType.DMA((N_DEV - 1, 2)),
            pltpu.SemaphoreType.DMA((N_DEV - 1, 2)),
            pltpu.SemaphoreType.DMA((N_DEV - 1,)),
            pltpu.SemaphoreType.DMA((N_DEV - 1,)),
        ],
        compiler_params=pltpu.CompilerParams(
            vmem_limit_bytes=63 * 1024 * 1024,
        ),
    )(x, w_mat)
